# baseline (speedup 1.0000x reference)
import functools
import jax
import jax.numpy as jnp
from jax import lax
from jax.experimental import pallas as pl
from jax.experimental.pallas import tpu as pltpu
from jax.experimental.pallas import tpu_sc as plsc

HID = 128
LANES = 16
NC = 2
NS = 16
NW = NC * NS
NBUF = 10
LOOKAHEAD = 5
ROWS = 64


def _make_kernel(B, S):
    assert B % NW == 0
    bs_per_w = B // NW
    assert bs_per_w % ROWS == 0
    halves = bs_per_w // ROWS
    nchunks = S * halves
    assert nchunks % NBUF == 0

    mesh = plsc.VectorSubcoreMesh(core_axis_name="c", subcore_axis_name="s")

    @functools.partial(
        pl.kernel,
        mesh=mesh,
        out_type=jax.ShapeDtypeStruct((S, B, HID), jnp.float32),
        scratch_types=[
            pltpu.VMEM((S, bs_per_w), jnp.int32),
            pltpu.VMEM((S, HID), jnp.float32),
        ] + [pltpu.VMEM((ROWS, HID), jnp.float32)
             for _ in range(NBUF)] + [
            pltpu.SemaphoreType.DMA((NBUF,)),
            pltpu.SemaphoreType.DMA((NBUF,)),
        ],
    )
    def k(srct_hbm, tok_hbm, pos_hbm, out_hbm, idx_v, pos_v, *rest):
        bufs = list(rest[:NBUF])
        gsem, ssem = rest[NBUF], rest[NBUF + 1]
        c = lax.axis_index("c")
        s = lax.axis_index("s")
        wid = s * NC + c
        base = wid * bs_per_w

        pltpu.sync_copy(srct_hbm.at[:, pl.ds(base, bs_per_w)], idx_v)
        pltpu.sync_copy(pos_hbm, pos_v)

        scale = jnp.float32(HID ** 0.5)

        def idx_slice(g):
            return idx_v.at[g // halves, pl.ds((g % halves) * ROWS, ROWS)]

        def out_slice(g):
            return out_hbm.at[g // halves,
                              pl.ds(base + (g % halves) * ROWS, ROWS)]

        def issue_gather(g, b):
            pltpu.async_copy(tok_hbm.at[idx_slice(g)], bufs[b], gsem.at[b])

        def wait_gather(g, b):
            pltpu.make_async_copy(tok_hbm.at[idx_slice(g)], bufs[b],
                                  gsem.at[b]).wait()

        def wait_store(g, b):
            pltpu.make_async_copy(bufs[b], out_slice(g), ssem.at[b]).wait()

        for p in range(LOOKAHEAD):
            issue_gather(jnp.int32(p), p)

        def outer_body(i, carry):
            for b in range(NBUF):
                g = i * NBUF + b
                nb = (b + LOOKAHEAD) % NBUF

                @pl.when(g + LOOKAHEAD < nchunks)
                def _():
                    @pl.when(g >= NBUF - LOOKAHEAD)
                    def _():
                        wait_store(g - (NBUF - LOOKAHEAD), nb)
                    issue_gather(g + LOOKAHEAD, nb)

                wait_gather(g, b)

                buf = bufs[b]
                pvec = [pos_v[g // halves, pl.ds(j * LANES, LANES)]
                        for j in range(HID // LANES)]

                @plsc.parallel_loop(0, ROWS, unroll=2)
                def fma(r):
                    for j in range(HID // LANES):
                        col = pl.ds(j * LANES, LANES)
                        buf[r, col] = buf[r, col] * scale + pvec[j]

                pltpu.async_copy(buf, out_slice(g), ssem.at[b])
            return carry

        lax.fori_loop(0, nchunks // NBUF, outer_body, 0)

        for j in range(NBUF):
            g = nchunks - NBUF + j
            wait_store(g, g % NBUF)

    return k


def kernel(src, tok_table, pos_table):
    B, S = src.shape
    src_t = jnp.transpose(jnp.asarray(src, jnp.int32))
    out_sb = _make_kernel(B, S)(src_t, tok_table, pos_table[:S])
    return jnp.transpose(out_sb, (1, 0, 2))


# speedup vs baseline: 1.0090x; 1.0090x over previous
import functools
import jax
import jax.numpy as jnp
from jax import lax
from jax.experimental import pallas as pl
from jax.experimental.pallas import tpu as pltpu
from jax.experimental.pallas import tpu_sc as plsc

HID = 128
LANES = 16
NC = 2
NS = 16
NW = NC * NS
NBUF = 5


def _make_kernel(B, S):
    assert B % NW == 0
    bs_per_w = B // NW
    assert bs_per_w % 8 == 0
    nchunks = S
    assert nchunks % NBUF == 0

    mesh = plsc.VectorSubcoreMesh(core_axis_name="c", subcore_axis_name="s")

    @functools.partial(
        pl.kernel,
        mesh=mesh,
        out_type=jax.ShapeDtypeStruct((S, B, HID), jnp.float32),
        scratch_types=[
            pltpu.VMEM((S, bs_per_w), jnp.int32),
            pltpu.VMEM((S, HID), jnp.float32),
        ] + [pltpu.VMEM((bs_per_w, HID), jnp.float32)
             for _ in range(NBUF)] + [
            pltpu.SemaphoreType.DMA((NBUF,)),
            pltpu.SemaphoreType.DMA((NBUF,)),
        ],
    )
    def k(srct_hbm, tok_hbm, pos_hbm, out_hbm, idx_v, pos_v, b0, b1, b2, b3,
          b4, gsem, ssem):
        bufs = [b0, b1, b2, b3, b4]
        c = lax.axis_index("c")
        s = lax.axis_index("s")
        wid = s * NC + c
        base = wid * bs_per_w

        pltpu.sync_copy(srct_hbm.at[:, pl.ds(base, bs_per_w)], idx_v)
        pltpu.sync_copy(pos_hbm, pos_v)

        scale = jnp.float32(HID ** 0.5)

        def issue_gather(g, b):
            pltpu.async_copy(tok_hbm.at[idx_v.at[g]], bufs[b], gsem.at[b])

        def wait_gather(g, b):
            pltpu.make_async_copy(tok_hbm.at[idx_v.at[g]], bufs[b],
                                  gsem.at[b]).wait()

        def out_slice(g):
            return out_hbm.at[g, pl.ds(base, bs_per_w)]

        def wait_store(g, b):
            pltpu.make_async_copy(bufs[b], out_slice(g), ssem.at[b]).wait()

        issue_gather(jnp.int32(0), 0)
        issue_gather(jnp.int32(1), 1)
        issue_gather(jnp.int32(2), 2)

        def outer_body(i, carry):
            for b in range(NBUF):
                g = i * NBUF + b
                nb = (b + 3) % NBUF

                @pl.when(g + 3 < nchunks)
                def _():
                    @pl.when(g >= 2)
                    def _():
                        wait_store(g - 2, nb)
                    issue_gather(g + 3, nb)

                wait_gather(g, b)

                buf = bufs[b]
                pvec = [pos_v[g, pl.ds(j * LANES, LANES)]
                        for j in range(HID // LANES)]

                @plsc.parallel_loop(0, bs_per_w, unroll=2)
                def fma(r):
                    for j in range(HID // LANES):
                        col = pl.ds(j * LANES, LANES)
                        buf[r, col] = buf[r, col] * scale + pvec[j]

                pltpu.async_copy(buf, out_slice(g), ssem.at[b])
            return carry

        lax.fori_loop(0, nchunks // NBUF, outer_body, 0)

        for j in range(NBUF):
            g = nchunks - NBUF + j
            wait_store(g, g % NBUF)

    return k


def kernel(src, tok_table, pos_table):
    B, S = src.shape
    src_t = jnp.transpose(jnp.asarray(src, jnp.int32))
    out_sb = _make_kernel(B, S)(src_t, tok_table, pos_table[:S])
    return jnp.transpose(out_sb, (1, 0, 2))
